# Initial kernel scaffold; baseline (speedup 1.0000x reference)
import jax
import jax.numpy as jnp
from jax.experimental import pallas as pl


def kernel(var_node_features, con_node_features, edge_index_var, edge_index_con, edge_features_var, edge_features_con, num_nodes_var, num_nodes_con, params):
    raise NotImplementedError("write your pallas kernel here")


# speedup vs baseline: 1.0427x; 1.0427x over previous
import functools

import jax
import jax.numpy as jnp
from jax import lax
from jax.experimental import pallas as pl
from jax.experimental.pallas import tpu as pltpu
from jax.experimental.pallas import tpu_sc as plsc

F32 = jnp.float32
H = 128
NV = 5000
E = 160000
NCORE = 2
NSUB = 16
CHUNK = 128
NCHUNK = 40
EPW = CHUNK * NCHUNK
EPAD = NCORE * NSUB * EPW
ACCR = 5120
MTILE = 512
NMT = EPAD // MTILE
BNEPS = 1e-5


def _dotf(a, b):
    return jnp.dot(a, b, preferred_element_type=F32)


def _enc_body(x_ref, w1_ref, b1_ref, w2_ref, b2_ref, o_ref):
    h = jnp.maximum(_dotf(x_ref[...], w1_ref[...]) + b1_ref[...], 0.0)
    o_ref[...] = _dotf(h, w2_ref[...]) + b2_ref[...]


def _encoder(x, p):
    return pl.pallas_call(
        _enc_body,
        out_shape=jax.ShapeDtypeStruct((NV, H), F32),
    )(x, p["l1"]["W"], p["l1"]["b"].reshape(1, H),
      p["l2"]["W"], p["l2"]["b"].reshape(1, H))


def _proj_body(xs_ref, xt_ref, ws_ref, wt_ref, os_ref, ot_ref):
    os_ref[...] = _dotf(xs_ref[...], ws_ref[...])
    ot_ref[...] = _dotf(xt_ref[...], wt_ref[...])


def _proj(x_src, x_tgt, w_src, w_tgt):
    return pl.pallas_call(
        _proj_body,
        out_shape=[jax.ShapeDtypeStruct((NV, H), F32),
                   jax.ShapeDtypeStruct((NV, H), F32)],
    )(x_src, x_tgt, w_src, w_tgt)


def _edge_z(ef, w1, b1, w2, b2):
    h1 = jnp.maximum(ef * w1 + b1, 0.0)
    return jnp.maximum(_dotf(h1, w2) + b2, 0.0)


def _estats_body(ef_ref, w1_ref, b1_ref, w2_ref, b2_ref, o_ref):
    i = pl.program_id(0)
    z = _edge_z(ef_ref[...], w1_ref[...], b1_ref[...], w2_ref[...], b2_ref[...])
    rid = lax.broadcasted_iota(jnp.int32, (MTILE, 1), 0) + i * MTILE
    msk = rid < E
    s0 = jnp.sum(jnp.where(msk, z, 0.0), axis=0, keepdims=True)
    s1 = jnp.sum(jnp.where(msk, z * z, 0.0), axis=0, keepdims=True)

    @pl.when(i == 0)
    def _():
        o_ref[...] = jnp.zeros_like(o_ref)

    o_ref[...] += jnp.concatenate([s0, s1], axis=0)


def _bn_affine(stats, gamma, beta):
    mean = stats[0:1] / E
    var = stats[1:2] / E - mean * mean
    a = gamma * lax.rsqrt(var + BNEPS)
    return a, beta - mean * a


def _eq_body(ef_ref, w1_ref, b1_ref, w2_ref, b2_ref, st_ref, g_ref, bt_ref,
             we_ref, bn1_ref, o_ref):
    z = _edge_z(ef_ref[...], w1_ref[...], b1_ref[...], w2_ref[...], b2_ref[...])
    a, c = _bn_affine(st_ref[...], g_ref[...], bt_ref[...])
    o_ref[...] = _dotf(z * a + c, we_ref[...]) + bn1_ref[...]


def _edge_q(ef_pad, enc_p, w_e, b_nn1):
    w1 = enc_p["l1"]["W"]
    b1 = enc_p["l1"]["b"].reshape(1, H)
    w2 = enc_p["l2"]["W"]
    b2 = enc_p["l2"]["b"].reshape(1, H)
    gamma = enc_p["bn"]["gamma"].reshape(1, H)
    beta = enc_p["bn"]["beta"].reshape(1, H)
    wspec = pl.BlockSpec((1, H), lambda i: (0, 0))
    wwspec = pl.BlockSpec((H, H), lambda i: (0, 0))
    efspec = pl.BlockSpec((MTILE, 1), lambda i: (i, 0))
    stats = pl.pallas_call(
        _estats_body,
        grid=(NMT,),
        in_specs=[efspec, wspec, wspec, wwspec, wspec],
        out_specs=pl.BlockSpec((2, H), lambda i: (0, 0)),
        out_shape=jax.ShapeDtypeStruct((2, H), F32),
    )(ef_pad, w1, b1, w2, b2)
    return pl.pallas_call(
        _eq_body,
        grid=(NMT,),
        in_specs=[efspec, wspec, wspec, wwspec, wspec,
                  pl.BlockSpec((2, H), lambda i: (0, 0)), wspec, wspec,
                  wwspec, wspec],
        out_specs=pl.BlockSpec((MTILE, H), lambda i: (i, 0)),
        out_shape=jax.ShapeDtypeStruct((EPAD, H), F32),
    )(ef_pad, w1, b1, w2, b2, stats, gamma, beta, w_e, b_nn1.reshape(1, H))


def _msg_body(g_ref, w2_ref, b2_ref, z_ref, st_ref):
    i = pl.program_id(0)
    h = jnp.maximum(g_ref[...], 0.0)
    z = jnp.maximum(_dotf(h, w2_ref[...]) + b2_ref[...], 0.0)
    z_ref[...] = z
    rid = lax.broadcasted_iota(jnp.int32, (MTILE, 1), 0) + i * MTILE
    msk = rid < E
    s0 = jnp.sum(jnp.where(msk, z, 0.0), axis=0, keepdims=True)
    s1 = jnp.sum(jnp.where(msk, z * z, 0.0), axis=0, keepdims=True)

    @pl.when(i == 0)
    def _():
        st_ref[...] = jnp.zeros_like(st_ref)

    st_ref[...] += jnp.concatenate([s0, s1], axis=0)


def _message(g, w2, b2):
    return pl.pallas_call(
        _msg_body,
        grid=(NMT,),
        in_specs=[pl.BlockSpec((MTILE, H), lambda i: (i, 0)),
                  pl.BlockSpec((H, H), lambda i: (0, 0)),
                  pl.BlockSpec((1, H), lambda i: (0, 0))],
        out_specs=[pl.BlockSpec((MTILE, H), lambda i: (i, 0)),
                   pl.BlockSpec((2, H), lambda i: (0, 0))],
        out_shape=[jax.ShapeDtypeStruct((EPAD, H), F32),
                   jax.ShapeDtypeStruct((2, H), F32)],
    )(g, w2, b2)


def _post_body(acc_ref, cnt_ref, st_ref, g_ref, bt_ref, o_ref):
    s = acc_ref[0, :NV, :] + acc_ref[1, :NV, :]
    cnt = cnt_ref[0, :NV, 0:1] + cnt_ref[1, :NV, 0:1]
    a, c = _bn_affine(st_ref[...], g_ref[...], bt_ref[...])
    o_ref[...] = jnp.maximum((s * a + cnt * c) / jnp.maximum(cnt, 1.0), 0.0)


def _post(accp, cntp, stats, bn_p):
    return pl.pallas_call(
        _post_body,
        out_shape=jax.ShapeDtypeStruct((NV, H), F32),
    )(accp, cntp, stats, bn_p["gamma"].reshape(1, H), bn_p["beta"].reshape(1, H))


def _head_body(x_ref, w1_ref, b1_ref, w2_ref, b2_ref, w3_ref, b3_ref,
               w4_ref, b4_ref, o_ref):
    h = jnp.maximum(_dotf(x_ref[...], w1_ref[...]) + b1_ref[...], 0.0)
    h = jnp.maximum(_dotf(h, w2_ref[...]) + b2_ref[...], 0.0)
    h = jnp.maximum(_dotf(h, w3_ref[...]) + b3_ref[...], 0.0)
    o = _dotf(h, w4_ref[...]) + b4_ref[...]
    m = jnp.max(o, axis=-1, keepdims=True)
    lse = m + jnp.log(jnp.sum(jnp.exp(o - m), axis=-1, keepdims=True))
    o_ref[...] = o - lse


def _head(x, params):
    args = [x]
    for name in ("lin1", "lin2", "lin3", "lin4"):
        args.append(params[name]["W"])
        args.append(params[name]["b"].reshape(1, -1))
    return pl.pallas_call(
        _head_body,
        out_shape=jax.ShapeDtypeStruct((NV, 2), F32),
    )(*args)


def _gather_body(ts_hbm, tt_hbm, idxs_hbm, idxt_hbm, q_hbm, g_hbm,
               idx_s, idx_t, rows_s, rows_t, qv, sem):
    c = lax.axis_index("c")
    s = lax.axis_index("s")
    base = (c * NSUB + s) * EPW
    pltpu.sync_copy(idxs_hbm.at[c, s], idx_s)
    pltpu.sync_copy(idxt_hbm.at[c, s], idx_t)

    def body(j, carry):
        pltpu.sync_copy(q_hbm.at[pl.ds(base + j * CHUNK, CHUNK)], qv)
        cp_a = pltpu.async_copy(ts_hbm.at[idx_s.at[j]], rows_s, sem)
        cp_b = pltpu.async_copy(tt_hbm.at[idx_t.at[j]], rows_t, sem)
        cp_a.wait()
        cp_b.wait()

        def inner(r, carry2):
            for gidx in range(H // 16):
                sl = pl.ds(gidx * 16, 16)
                rows_s[r, sl] = rows_s[r, sl] + rows_t[r, sl] + qv[r, sl]
            return carry2

        lax.fori_loop(0, CHUNK, inner, 0)
        pltpu.sync_copy(rows_s, g_hbm.at[pl.ds(base + j * CHUNK, CHUNK)])
        return carry

    lax.fori_loop(0, NCHUNK, body, 0)


def _scatter_body(z_hbm, idxt_hbm, zeros_hbm, out_hbm, idx_t, zv, acc):
    c = lax.axis_index("c")
    s = lax.axis_index("s")

    @pl.when(s == 0)
    def _():
        pltpu.sync_copy(zeros_hbm, acc)

    plsc.subcore_barrier()
    base = (c * NSUB + s) * EPW
    pltpu.sync_copy(idxt_hbm.at[c, s], idx_t)

    def body(j, carry):
        pltpu.sync_copy(z_hbm.at[pl.ds(base + j * CHUNK, CHUNK)], zv)
        pltpu.sync_copy(zv, acc.at[idx_t.at[j]], add=True)
        return carry

    lax.fori_loop(0, NCHUNK, body, 0)
    plsc.subcore_barrier()

    @pl.when(s == 0)
    def _():
        pltpu.sync_copy(acc, out_hbm.at[c])


def _counts_body(idxt_hbm, zeros_hbm, ones_hbm, out_hbm, idx_t, onev, acc):
    c = lax.axis_index("c")
    s = lax.axis_index("s")

    @pl.when(s == 0)
    def _():
        pltpu.sync_copy(zeros_hbm, acc)

    pltpu.sync_copy(ones_hbm, onev)
    plsc.subcore_barrier()
    pltpu.sync_copy(idxt_hbm.at[c, s], idx_t)

    def body(j, carry):
        pltpu.sync_copy(onev, acc.at[idx_t.at[j]], add=True)
        return carry

    lax.fori_loop(0, NCHUNK, body, 0)
    plsc.subcore_barrier()

    @pl.when(s == 0)
    def _():
        pltpu.sync_copy(acc, out_hbm.at[c])


@functools.lru_cache(maxsize=None)
def _sc_kernels():
    mesh = plsc.VectorSubcoreMesh(core_axis_name="c", subcore_axis_name="s",
                                  num_cores=NCORE, num_subcores=NSUB)
    gather = pl.kernel(
        _gather_body,
        out_type=jax.ShapeDtypeStruct((EPAD, H), F32),
        mesh=mesh,
        scratch_types=[
            pltpu.VMEM((NCHUNK, CHUNK), jnp.int32),
            pltpu.VMEM((NCHUNK, CHUNK), jnp.int32),
            pltpu.VMEM((CHUNK, H), F32),
            pltpu.VMEM((CHUNK, H), F32),
            pltpu.VMEM((CHUNK, H), F32),
            pltpu.SemaphoreType.DMA,
        ],
    )
    scatter = pl.kernel(
        _scatter_body,
        out_type=jax.ShapeDtypeStruct((NCORE, ACCR, H), F32),
        mesh=mesh,
        scratch_types=[
            pltpu.VMEM((NCHUNK, CHUNK), jnp.int32),
            pltpu.VMEM((CHUNK, H), F32),
            pltpu.VMEM_SHARED((ACCR, H), F32),
        ],
    )
    counts = pl.kernel(
        _counts_body,
        out_type=jax.ShapeDtypeStruct((NCORE, ACCR, H), F32),
        mesh=mesh,
        scratch_types=[
            pltpu.VMEM((NCHUNK, CHUNK), jnp.int32),
            pltpu.VMEM((CHUNK, H), F32),
            pltpu.VMEM_SHARED((ACCR, H), F32),
        ],
    )
    return gather, scatter, counts


def _gather_sc(ts, tt, idxs, idxt, q):
    return _sc_kernels()[0](ts, tt, idxs, idxt, q)


def _scatter_sc(z, idxt):
    return _sc_kernels()[1](z, idxt, jnp.zeros((ACCR, H), F32))


def _counts_sc(idxt):
    return _sc_kernels()[2](idxt, jnp.zeros((ACCR, H), F32),
                            jnp.ones((CHUNK, H), F32))


def _prep_idx(ei):
    pad = EPAD - E
    src = ei[0].astype(jnp.int32)
    tgt = ei[1].astype(jnp.int32)
    shape = (NCORE, NSUB, NCHUNK, CHUNK)
    src_g = jnp.pad(src, (0, pad)).reshape(shape)
    tgt_g = jnp.pad(tgt, (0, pad)).reshape(shape)
    tgt_s = jnp.pad(tgt, (0, pad), constant_values=NV).reshape(shape)
    return src_g, tgt_g, tgt_s


def _bipartite(x_src, x_tgt, src_g, tgt_g, tgt_s, cntp, q, p):
    w1 = p["nn"]["l1"]["W"]
    p_src, p_tgt = _proj(x_src, x_tgt, w1[H:2 * H], w1[0:H])
    g = _gather_sc(p_src, p_tgt, src_g, tgt_g, q)
    z, stats = _message(g, p["nn"]["l2"]["W"], p["nn"]["l2"]["b"].reshape(1, H))
    accp = _scatter_sc(z, tgt_s)
    return _post(accp, cntp, stats, p["nn"]["bn"])


def kernel(var_node_features, con_node_features, edge_index_var,
           edge_index_con, edge_features_var, edge_features_con,
           num_nodes_var, num_nodes_con, params):
    del num_nodes_var, num_nodes_con
    src_gv, tgt_gv, tgt_sv = _prep_idx(edge_index_var)
    src_gc, tgt_gc, tgt_sc_ = _prep_idx(edge_index_con)
    ef_var = jnp.pad(edge_features_var.astype(F32), ((0, EPAD - E), (0, 0)))
    ef_con = jnp.pad(edge_features_con.astype(F32), ((0, EPAD - E), (0, 0)))

    xv = _encoder(var_node_features.astype(F32), params["var_enc"])
    xc = _encoder(con_node_features.astype(F32), params["con_enc"])

    cntp_c = _counts_sc(tgt_sv)
    cntp_v = _counts_sc(tgt_sc_)

    qs_var = [_edge_q(ef_var, p["edge_encoder"], p["nn"]["l1"]["W"][2 * H:],
                      p["nn"]["l1"]["b"]) for p in params["layers_var"]]
    qs_con = [_edge_q(ef_con, p["edge_encoder"], p["nn"]["l1"]["W"][2 * H:],
                      p["nn"]["l1"]["b"]) for p in params["layers_con"]]

    x_var = [xv]
    x_con = [xc]
    for i in range(2):
        x_con.append(_bipartite(x_var[-1], x_con[-1], src_gv, tgt_gv, tgt_sv,
                                cntp_c, qs_var[i], params["layers_var"][i]))
        x_var.append(_bipartite(x_con[-1], x_var[-1], src_gc, tgt_gc, tgt_sc_,
                                cntp_v, qs_con[i], params["layers_con"][i]))

    x = jnp.concatenate(x_var, axis=-1)
    return _head(x, params)


# speedup vs baseline: 1.1030x; 1.0578x over previous
import functools

import jax
import jax.numpy as jnp
from jax import lax
from jax.experimental import pallas as pl
from jax.experimental.pallas import tpu as pltpu
from jax.experimental.pallas import tpu_sc as plsc

F32 = jnp.float32
H = 128
NV = 5000
E = 160000
NCORE = 2
NSUB = 16
CHUNK = 128
NCHUNK = 40
EPW = CHUNK * NCHUNK
EPAD = NCORE * NSUB * EPW
ACCR = 5120
MTILE = 512
NMT = EPAD // MTILE
BNEPS = 1e-5


def _dotf(a, b):
    return jnp.dot(a, b, preferred_element_type=F32)


def _enc_body(x_ref, w1_ref, b1_ref, w2_ref, b2_ref, o_ref):
    h = jnp.maximum(_dotf(x_ref[...], w1_ref[...]) + b1_ref[...], 0.0)
    o_ref[...] = _dotf(h, w2_ref[...]) + b2_ref[...]


def _encoder(x, p):
    return pl.pallas_call(
        _enc_body,
        out_shape=jax.ShapeDtypeStruct((NV, H), F32),
    )(x, p["l1"]["W"], p["l1"]["b"].reshape(1, H),
      p["l2"]["W"], p["l2"]["b"].reshape(1, H))


def _proj_body(xs_ref, xt_ref, ws_ref, wt_ref, os_ref, ot_ref):
    os_ref[...] = _dotf(xs_ref[...], ws_ref[...])
    ot_ref[...] = _dotf(xt_ref[...], wt_ref[...])


def _proj(x_src, x_tgt, w_src, w_tgt):
    return pl.pallas_call(
        _proj_body,
        out_shape=[jax.ShapeDtypeStruct((NV, H), F32),
                   jax.ShapeDtypeStruct((NV, H), F32)],
    )(x_src, x_tgt, w_src, w_tgt)


def _edge_z(ef, w1, b1, w2, b2):
    h1 = jnp.maximum(ef * w1 + b1, 0.0)
    return jnp.maximum(_dotf(h1, w2) + b2, 0.0)


def _estats_body(ef_ref, w1_ref, b1_ref, w2_ref, b2_ref, o_ref):
    i = pl.program_id(0)
    z = _edge_z(ef_ref[...], w1_ref[...], b1_ref[...], w2_ref[...], b2_ref[...])
    rid = lax.broadcasted_iota(jnp.int32, (MTILE, 1), 0) + i * MTILE
    msk = rid < E
    s0 = jnp.sum(jnp.where(msk, z, 0.0), axis=0, keepdims=True)
    s1 = jnp.sum(jnp.where(msk, z * z, 0.0), axis=0, keepdims=True)

    @pl.when(i == 0)
    def _():
        o_ref[...] = jnp.zeros_like(o_ref)

    o_ref[...] += jnp.concatenate([s0, s1], axis=0)


def _bn_affine(stats, gamma, beta):
    mean = stats[0:1] / E
    var = stats[1:2] / E - mean * mean
    a = gamma * lax.rsqrt(var + BNEPS)
    return a, beta - mean * a


def _eq_body(ef_ref, w1_ref, b1_ref, w2_ref, b2_ref, st_ref, g_ref, bt_ref,
             we_ref, bn1_ref, o_ref):
    z = _edge_z(ef_ref[...], w1_ref[...], b1_ref[...], w2_ref[...], b2_ref[...])
    a, c = _bn_affine(st_ref[...], g_ref[...], bt_ref[...])
    o_ref[...] = _dotf(z * a + c, we_ref[...]) + bn1_ref[...]


def _edge_q(ef_pad, enc_p, w_e, b_nn1):
    w1 = enc_p["l1"]["W"]
    b1 = enc_p["l1"]["b"].reshape(1, H)
    w2 = enc_p["l2"]["W"]
    b2 = enc_p["l2"]["b"].reshape(1, H)
    gamma = enc_p["bn"]["gamma"].reshape(1, H)
    beta = enc_p["bn"]["beta"].reshape(1, H)
    wspec = pl.BlockSpec((1, H), lambda i: (0, 0))
    wwspec = pl.BlockSpec((H, H), lambda i: (0, 0))
    efspec = pl.BlockSpec((MTILE, 1), lambda i: (i, 0))
    stats = pl.pallas_call(
        _estats_body,
        grid=(NMT,),
        in_specs=[efspec, wspec, wspec, wwspec, wspec],
        out_specs=pl.BlockSpec((2, H), lambda i: (0, 0)),
        out_shape=jax.ShapeDtypeStruct((2, H), F32),
    )(ef_pad, w1, b1, w2, b2)
    return pl.pallas_call(
        _eq_body,
        grid=(NMT,),
        in_specs=[efspec, wspec, wspec, wwspec, wspec,
                  pl.BlockSpec((2, H), lambda i: (0, 0)), wspec, wspec,
                  wwspec, wspec],
        out_specs=pl.BlockSpec((MTILE, H), lambda i: (i, 0)),
        out_shape=jax.ShapeDtypeStruct((EPAD, H), F32),
    )(ef_pad, w1, b1, w2, b2, stats, gamma, beta, w_e, b_nn1.reshape(1, H))


def _msg_body(g_ref, w2_ref, b2_ref, z_ref, st_ref):
    i = pl.program_id(0)
    h = jnp.maximum(g_ref[...], 0.0)
    z = jnp.maximum(_dotf(h, w2_ref[...]) + b2_ref[...], 0.0)
    z_ref[...] = z
    rid = lax.broadcasted_iota(jnp.int32, (MTILE, 1), 0) + i * MTILE
    msk = rid < E
    s0 = jnp.sum(jnp.where(msk, z, 0.0), axis=0, keepdims=True)
    s1 = jnp.sum(jnp.where(msk, z * z, 0.0), axis=0, keepdims=True)

    @pl.when(i == 0)
    def _():
        st_ref[...] = jnp.zeros_like(st_ref)

    st_ref[...] += jnp.concatenate([s0, s1], axis=0)


def _message(g, w2, b2):
    return pl.pallas_call(
        _msg_body,
        grid=(NMT,),
        in_specs=[pl.BlockSpec((MTILE, H), lambda i: (i, 0)),
                  pl.BlockSpec((H, H), lambda i: (0, 0)),
                  pl.BlockSpec((1, H), lambda i: (0, 0))],
        out_specs=[pl.BlockSpec((MTILE, H), lambda i: (i, 0)),
                   pl.BlockSpec((2, H), lambda i: (0, 0))],
        out_shape=[jax.ShapeDtypeStruct((EPAD, H), F32),
                   jax.ShapeDtypeStruct((2, H), F32)],
    )(g, w2, b2)


def _post_body(acc_ref, cnt_ref, st_ref, g_ref, bt_ref, o_ref):
    s = acc_ref[0, :NV, :] + acc_ref[1, :NV, :]
    cnt = cnt_ref[0, :NV, 0:1] + cnt_ref[1, :NV, 0:1]
    a, c = _bn_affine(st_ref[...], g_ref[...], bt_ref[...])
    o_ref[...] = jnp.maximum((s * a + cnt * c) / jnp.maximum(cnt, 1.0), 0.0)


def _post(accp, cntp, stats, bn_p):
    return pl.pallas_call(
        _post_body,
        out_shape=jax.ShapeDtypeStruct((NV, H), F32),
    )(accp, cntp, stats, bn_p["gamma"].reshape(1, H), bn_p["beta"].reshape(1, H))


def _head_body(x_ref, w1_ref, b1_ref, w2_ref, b2_ref, w3_ref, b3_ref,
               w4_ref, b4_ref, o_ref):
    h = jnp.maximum(_dotf(x_ref[...], w1_ref[...]) + b1_ref[...], 0.0)
    h = jnp.maximum(_dotf(h, w2_ref[...]) + b2_ref[...], 0.0)
    h = jnp.maximum(_dotf(h, w3_ref[...]) + b3_ref[...], 0.0)
    o = _dotf(h, w4_ref[...]) + b4_ref[...]
    m = jnp.max(o, axis=-1, keepdims=True)
    lse = m + jnp.log(jnp.sum(jnp.exp(o - m), axis=-1, keepdims=True))
    o_ref[...] = o - lse


def _head(x, params):
    args = [x]
    for name in ("lin1", "lin2", "lin3", "lin4"):
        args.append(params[name]["W"])
        args.append(params[name]["b"].reshape(1, -1))
    return pl.pallas_call(
        _head_body,
        out_shape=jax.ShapeDtypeStruct((NV, 2), F32),
    )(*args)


def _gather_body(ts_hbm, tt_hbm, idxs_hbm, idxt_hbm, q_hbm, g_hbm,
                 idx_s, idx_t, rs0, rs1, rt0, rt1, qv0, qv1,
                 lq0, lq1, lg0, lg1, ss0, ss1):
    c = lax.axis_index("c")
    s = lax.axis_index("s")
    base = (c * NSUB + s) * EPW
    pltpu.sync_copy(idxs_hbm.at[c, s], idx_s)
    pltpu.sync_copy(idxt_hbm.at[c, s], idx_t)
    bufs = ((rs0, rt0, qv0, lq0, lg0, ss0), (rs1, rt1, qv1, lq1, lg1, ss1))

    def body(k, carry):
        for b, (rs, rt, qv, lqs, lgs, ssem) in enumerate(bufs):
            j = 2 * k + b

            @pl.when(k > 0)
            def _():
                pltpu.make_async_copy(rs, g_hbm.at[pl.ds(0, CHUNK)],
                                      ssem).wait()

            pltpu.async_copy(q_hbm.at[pl.ds(base + j * CHUNK, CHUNK)], qv,
                             lqs)
            pltpu.async_copy(ts_hbm.at[idx_s.at[j]], rs, lgs)
            pltpu.async_copy(tt_hbm.at[idx_t.at[j]], rt, lgs)
        for b, (rs, rt, qv, lqs, lgs, ssem) in enumerate(bufs):
            j = 2 * k + b
            pltpu.make_async_copy(q_hbm.at[pl.ds(0, CHUNK)], qv, lqs).wait()
            pltpu.make_async_copy(ts_hbm.at[idx_s.at[j]], rs, lgs).wait()
            pltpu.make_async_copy(tt_hbm.at[idx_t.at[j]], rt, lgs).wait()

            def inner(r, carry2):
                for gidx in range(H // 16):
                    sl = pl.ds(gidx * 16, 16)
                    rs[r, sl] = rs[r, sl] + rt[r, sl] + qv[r, sl]
                return carry2

            lax.fori_loop(0, CHUNK, inner, 0)
            pltpu.async_copy(rs, g_hbm.at[pl.ds(base + j * CHUNK, CHUNK)],
                             ssem)
        return carry

    lax.fori_loop(0, NCHUNK // 2, body, 0)
    for rs, rt, qv, lqs, lgs, ssem in bufs:
        pltpu.make_async_copy(rs, g_hbm.at[pl.ds(0, CHUNK)], ssem).wait()


def _scatter_body(z_hbm, idxt_hbm, zeros_hbm, out_hbm, idx_t, zv0, zv1,
                  acc, lz0, lz1, sc0, sc1):
    c = lax.axis_index("c")
    s = lax.axis_index("s")

    @pl.when(s == 0)
    def _():
        pltpu.sync_copy(zeros_hbm, acc)

    plsc.subcore_barrier()
    base = (c * NSUB + s) * EPW
    pltpu.sync_copy(idxt_hbm.at[c, s], idx_t)
    bufs = ((zv0, lz0, sc0), (zv1, lz1, sc1))

    def body(k, carry):
        for b, (zv, lzs, scs) in enumerate(bufs):
            j = 2 * k + b

            @pl.when(k > 0)
            def _():
                pltpu.make_async_copy(zv, acc.at[idx_t.at[j]],
                                      scs).wait()

            pltpu.async_copy(z_hbm.at[pl.ds(base + j * CHUNK, CHUNK)], zv,
                             lzs)
        for b, (zv, lzs, scs) in enumerate(bufs):
            j = 2 * k + b
            pltpu.make_async_copy(z_hbm.at[pl.ds(0, CHUNK)], zv, lzs).wait()
            pltpu.async_copy(zv, acc.at[idx_t.at[j]], scs,
                             add=True)
        return carry

    lax.fori_loop(0, NCHUNK // 2, body, 0)
    for zv, lzs, scs in bufs:
        pltpu.make_async_copy(zv, acc.at[idx_t.at[0]],
                              scs).wait()
    plsc.subcore_barrier()

    @pl.when(s == 0)
    def _():
        pltpu.sync_copy(acc, out_hbm.at[c])


def _counts_body(idxt_hbm, zeros_hbm, ones_hbm, out_hbm, idx_t, onev, acc,
                 csem):
    c = lax.axis_index("c")
    s = lax.axis_index("s")

    @pl.when(s == 0)
    def _():
        pltpu.sync_copy(zeros_hbm, acc)

    pltpu.sync_copy(ones_hbm, onev)
    plsc.subcore_barrier()
    pltpu.sync_copy(idxt_hbm.at[c, s], idx_t)

    def body(j, carry):
        pltpu.async_copy(onev, acc.at[idx_t.at[j]], csem, add=True)
        return carry

    lax.fori_loop(0, NCHUNK, body, 0)

    def drain(j, carry):
        pltpu.make_async_copy(onev, acc.at[idx_t.at[0]], csem).wait()
        return carry

    lax.fori_loop(0, NCHUNK, drain, 0)
    plsc.subcore_barrier()

    @pl.when(s == 0)
    def _():
        pltpu.sync_copy(acc, out_hbm.at[c])


@functools.lru_cache(maxsize=None)
def _sc_kernels():
    mesh = plsc.VectorSubcoreMesh(core_axis_name="c", subcore_axis_name="s",
                                  num_cores=NCORE, num_subcores=NSUB)
    gather = pl.kernel(
        _gather_body,
        out_type=jax.ShapeDtypeStruct((EPAD, H), F32),
        mesh=mesh,
        scratch_types=[
            pltpu.VMEM((NCHUNK, CHUNK), jnp.int32),
            pltpu.VMEM((NCHUNK, CHUNK), jnp.int32),
            pltpu.VMEM((CHUNK, H), F32),
            pltpu.VMEM((CHUNK, H), F32),
            pltpu.VMEM((CHUNK, H), F32),
            pltpu.VMEM((CHUNK, H), F32),
            pltpu.VMEM((CHUNK, H), F32),
            pltpu.VMEM((CHUNK, H), F32),
            pltpu.SemaphoreType.DMA,
            pltpu.SemaphoreType.DMA,
            pltpu.SemaphoreType.DMA,
            pltpu.SemaphoreType.DMA,
            pltpu.SemaphoreType.DMA,
            pltpu.SemaphoreType.DMA,
        ],
    )
    scatter = pl.kernel(
        _scatter_body,
        out_type=jax.ShapeDtypeStruct((NCORE, ACCR, H), F32),
        mesh=mesh,
        scratch_types=[
            pltpu.VMEM((NCHUNK, CHUNK), jnp.int32),
            pltpu.VMEM((CHUNK, H), F32),
            pltpu.VMEM((CHUNK, H), F32),
            pltpu.VMEM_SHARED((ACCR, H), F32),
            pltpu.SemaphoreType.DMA,
            pltpu.SemaphoreType.DMA,
            pltpu.SemaphoreType.DMA,
            pltpu.SemaphoreType.DMA,
        ],
    )
    counts = pl.kernel(
        _counts_body,
        out_type=jax.ShapeDtypeStruct((NCORE, ACCR, H), F32),
        mesh=mesh,
        scratch_types=[
            pltpu.VMEM((NCHUNK, CHUNK), jnp.int32),
            pltpu.VMEM((CHUNK, H), F32),
            pltpu.VMEM_SHARED((ACCR, H), F32),
            pltpu.SemaphoreType.DMA,
        ],
    )
    return gather, scatter, counts


def _gather_sc(ts, tt, idxs, idxt, q):
    return _sc_kernels()[0](ts, tt, idxs, idxt, q)


def _scatter_sc(z, idxt):
    return _sc_kernels()[1](z, idxt, jnp.zeros((ACCR, H), F32))


def _counts_sc(idxt):
    return _sc_kernels()[2](idxt, jnp.zeros((ACCR, H), F32),
                            jnp.ones((CHUNK, H), F32))


def _prep_idx(ei):
    pad = EPAD - E
    src = ei[0].astype(jnp.int32)
    tgt = ei[1].astype(jnp.int32)
    shape = (NCORE, NSUB, NCHUNK, CHUNK)
    src_g = jnp.pad(src, (0, pad)).reshape(shape)
    tgt_g = jnp.pad(tgt, (0, pad)).reshape(shape)
    tgt_s = jnp.pad(tgt, (0, pad), constant_values=NV).reshape(shape)
    return src_g, tgt_g, tgt_s


def _bipartite(x_src, x_tgt, src_g, tgt_g, tgt_s, cntp, q, p):
    w1 = p["nn"]["l1"]["W"]
    p_src, p_tgt = _proj(x_src, x_tgt, w1[H:2 * H], w1[0:H])
    g = _gather_sc(p_src, p_tgt, src_g, tgt_g, q)
    z, stats = _message(g, p["nn"]["l2"]["W"], p["nn"]["l2"]["b"].reshape(1, H))
    accp = _scatter_sc(z, tgt_s)
    return _post(accp, cntp, stats, p["nn"]["bn"])


def kernel(var_node_features, con_node_features, edge_index_var,
           edge_index_con, edge_features_var, edge_features_con,
           num_nodes_var, num_nodes_con, params):
    del num_nodes_var, num_nodes_con
    src_gv, tgt_gv, tgt_sv = _prep_idx(edge_index_var)
    src_gc, tgt_gc, tgt_sc_ = _prep_idx(edge_index_con)
    ef_var = jnp.pad(edge_features_var.astype(F32), ((0, EPAD - E), (0, 0)))
    ef_con = jnp.pad(edge_features_con.astype(F32), ((0, EPAD - E), (0, 0)))

    xv = _encoder(var_node_features.astype(F32), params["var_enc"])
    xc = _encoder(con_node_features.astype(F32), params["con_enc"])

    cntp_c = _counts_sc(tgt_sv)
    cntp_v = _counts_sc(tgt_sc_)

    qs_var = [_edge_q(ef_var, p["edge_encoder"], p["nn"]["l1"]["W"][2 * H:],
                      p["nn"]["l1"]["b"]) for p in params["layers_var"]]
    qs_con = [_edge_q(ef_con, p["edge_encoder"], p["nn"]["l1"]["W"][2 * H:],
                      p["nn"]["l1"]["b"]) for p in params["layers_con"]]

    x_var = [xv]
    x_con = [xc]
    for i in range(2):
        x_con.append(_bipartite(x_var[-1], x_con[-1], src_gv, tgt_gv, tgt_sv,
                                cntp_c, qs_var[i], params["layers_var"][i]))
        x_var.append(_bipartite(x_con[-1], x_var[-1], src_gc, tgt_gc, tgt_sc_,
                                cntp_v, qs_con[i], params["layers_con"][i]))

    x = jnp.concatenate(x_var, axis=-1)
    return _head(x, params)


# speedup vs baseline: 1.3990x; 1.2684x over previous
import functools

import jax
import jax.numpy as jnp
from jax import lax
from jax.experimental import pallas as pl
from jax.experimental.pallas import tpu as pltpu
from jax.experimental.pallas import tpu_sc as plsc

F32 = jnp.float32
H = 128
NV = 5000
E = 160000
NCORE = 2
NSUB = 16
CHUNK = 128
NCHUNK = 40
EPW = CHUNK * NCHUNK
EPAD = NCORE * NSUB * EPW
ACCR = 5120
MTILE = 512
NMT = EPAD // MTILE
BNEPS = 1e-5


def _dotf(a, b):
    return jnp.dot(a, b, preferred_element_type=F32)


def _enc_body(x_ref, w1_ref, b1_ref, w2_ref, b2_ref, o_ref):
    h = jnp.maximum(_dotf(x_ref[...], w1_ref[...]) + b1_ref[...], 0.0)
    o_ref[...] = _dotf(h, w2_ref[...]) + b2_ref[...]


def _encoder(x, p):
    return pl.pallas_call(
        _enc_body,
        out_shape=jax.ShapeDtypeStruct((NV, H), F32),
    )(x, p["l1"]["W"], p["l1"]["b"].reshape(1, H),
      p["l2"]["W"], p["l2"]["b"].reshape(1, H))


def _proj_body(xs_ref, xt_ref, ws_ref, wt_ref, os_ref, ot_ref):
    os_ref[...] = _dotf(xs_ref[...], ws_ref[...])
    ot_ref[...] = _dotf(xt_ref[...], wt_ref[...])


def _proj(x_src, x_tgt, w_src, w_tgt):
    return pl.pallas_call(
        _proj_body,
        out_shape=[jax.ShapeDtypeStruct((NV, H), F32),
                   jax.ShapeDtypeStruct((NV, H), F32)],
    )(x_src, x_tgt, w_src, w_tgt)


def _edge_z(ef, w1, b1, w2, b2):
    h1 = jnp.maximum(ef * w1 + b1, 0.0)
    return jnp.maximum(_dotf(h1, w2) + b2, 0.0)


def _estats_body(ef_ref, w1_ref, b1_ref, w2_ref, b2_ref, o_ref):
    i = pl.program_id(0)
    z = _edge_z(ef_ref[...], w1_ref[...], b1_ref[...], w2_ref[...], b2_ref[...])
    rid = lax.broadcasted_iota(jnp.int32, (MTILE, 1), 0) + i * MTILE
    msk = rid < E
    s0 = jnp.sum(jnp.where(msk, z, 0.0), axis=0, keepdims=True)
    s1 = jnp.sum(jnp.where(msk, z * z, 0.0), axis=0, keepdims=True)

    @pl.when(i == 0)
    def _():
        o_ref[...] = jnp.zeros_like(o_ref)

    o_ref[...] += jnp.concatenate([s0, s1], axis=0)


def _bn_affine(stats, gamma, beta):
    mean = stats[0:1] / E
    var = stats[1:2] / E - mean * mean
    a = gamma * lax.rsqrt(var + BNEPS)
    return a, beta - mean * a


def _eq_body(ef_ref, w1_ref, b1_ref, w2_ref, b2_ref, st_ref, g_ref, bt_ref,
             we_ref, bn1_ref, o_ref):
    z = _edge_z(ef_ref[...], w1_ref[...], b1_ref[...], w2_ref[...], b2_ref[...])
    a, c = _bn_affine(st_ref[...], g_ref[...], bt_ref[...])
    o_ref[...] = _dotf(z * a + c, we_ref[...]) + bn1_ref[...]


def _edge_q(ef_pad, enc_p, w_e, b_nn1):
    w1 = enc_p["l1"]["W"]
    b1 = enc_p["l1"]["b"].reshape(1, H)
    w2 = enc_p["l2"]["W"]
    b2 = enc_p["l2"]["b"].reshape(1, H)
    gamma = enc_p["bn"]["gamma"].reshape(1, H)
    beta = enc_p["bn"]["beta"].reshape(1, H)
    wspec = pl.BlockSpec((1, H), lambda i: (0, 0))
    wwspec = pl.BlockSpec((H, H), lambda i: (0, 0))
    efspec = pl.BlockSpec((MTILE, 1), lambda i: (i, 0))
    stats = pl.pallas_call(
        _estats_body,
        grid=(NMT,),
        in_specs=[efspec, wspec, wspec, wwspec, wspec],
        out_specs=pl.BlockSpec((2, H), lambda i: (0, 0)),
        out_shape=jax.ShapeDtypeStruct((2, H), F32),
    )(ef_pad, w1, b1, w2, b2)
    return pl.pallas_call(
        _eq_body,
        grid=(NMT,),
        in_specs=[efspec, wspec, wspec, wwspec, wspec,
                  pl.BlockSpec((2, H), lambda i: (0, 0)), wspec, wspec,
                  wwspec, wspec],
        out_specs=pl.BlockSpec((MTILE, H), lambda i: (i, 0)),
        out_shape=jax.ShapeDtypeStruct((EPAD, H), F32),
    )(ef_pad, w1, b1, w2, b2, stats, gamma, beta, w_e, b_nn1.reshape(1, H))


def _msg_body(g_ref, q_ref, w2_ref, b2_ref, z_ref, st_ref):
    i = pl.program_id(0)
    h = jnp.maximum(g_ref[...] + q_ref[...], 0.0)
    z = jnp.maximum(_dotf(h, w2_ref[...]) + b2_ref[...], 0.0)
    z_ref[...] = z
    rid = lax.broadcasted_iota(jnp.int32, (MTILE, 1), 0) + i * MTILE
    msk = rid < E
    s0 = jnp.sum(jnp.where(msk, z, 0.0), axis=0, keepdims=True)
    s1 = jnp.sum(jnp.where(msk, z * z, 0.0), axis=0, keepdims=True)

    @pl.when(i == 0)
    def _():
        st_ref[...] = jnp.zeros_like(st_ref)

    st_ref[...] += jnp.concatenate([s0, s1], axis=0)


def _message(g, q, w2, b2):
    return pl.pallas_call(
        _msg_body,
        grid=(NMT,),
        in_specs=[pl.BlockSpec((MTILE, H), lambda i: (i, 0)),
                  pl.BlockSpec((MTILE, H), lambda i: (i, 0)),
                  pl.BlockSpec((H, H), lambda i: (0, 0)),
                  pl.BlockSpec((1, H), lambda i: (0, 0))],
        out_specs=[pl.BlockSpec((MTILE, H), lambda i: (i, 0)),
                   pl.BlockSpec((2, H), lambda i: (0, 0))],
        out_shape=[jax.ShapeDtypeStruct((EPAD, H), F32),
                   jax.ShapeDtypeStruct((2, H), F32)],
    )(g, q, w2, b2)


def _post_body(acc_ref, cnt_ref, st_ref, g_ref, bt_ref, o_ref):
    s = acc_ref[0, :NV, :] + acc_ref[1, :NV, :]
    cnt = cnt_ref[0, :NV, 0:1] + cnt_ref[1, :NV, 0:1]
    a, c = _bn_affine(st_ref[...], g_ref[...], bt_ref[...])
    o_ref[...] = jnp.maximum((s * a + cnt * c) / jnp.maximum(cnt, 1.0), 0.0)


def _post(accp, cntp, stats, bn_p):
    return pl.pallas_call(
        _post_body,
        out_shape=jax.ShapeDtypeStruct((NV, H), F32),
    )(accp, cntp, stats, bn_p["gamma"].reshape(1, H), bn_p["beta"].reshape(1, H))


def _head_body(x_ref, w1_ref, b1_ref, w2_ref, b2_ref, w3_ref, b3_ref,
               w4_ref, b4_ref, o_ref):
    h = jnp.maximum(_dotf(x_ref[...], w1_ref[...]) + b1_ref[...], 0.0)
    h = jnp.maximum(_dotf(h, w2_ref[...]) + b2_ref[...], 0.0)
    h = jnp.maximum(_dotf(h, w3_ref[...]) + b3_ref[...], 0.0)
    o = _dotf(h, w4_ref[...]) + b4_ref[...]
    m = jnp.max(o, axis=-1, keepdims=True)
    lse = m + jnp.log(jnp.sum(jnp.exp(o - m), axis=-1, keepdims=True))
    o_ref[...] = o - lse


def _head(x, params):
    args = [x]
    for name in ("lin1", "lin2", "lin3", "lin4"):
        args.append(params[name]["W"])
        args.append(params[name]["b"].reshape(1, -1))
    return pl.pallas_call(
        _head_body,
        out_shape=jax.ShapeDtypeStruct((NV, 2), F32),
    )(*args)


def _gather_body(ts_hbm, tt_hbm, idxs_hbm, idxt_hbm, g_hbm,
                 idx_s, idx_t, rs0, rs1, rt0, rt1,
                 lg0, lg1, ss0, ss1):
    c = lax.axis_index("c")
    s = lax.axis_index("s")
    base = (c * NSUB + s) * EPW
    pltpu.sync_copy(idxs_hbm.at[c, s], idx_s)
    pltpu.sync_copy(idxt_hbm.at[c, s], idx_t)
    bufs = ((rs0, rt0, lg0, ss0), (rs1, rt1, lg1, ss1))

    def body(k, carry):
        for b, (rs, rt, lgs, ssem) in enumerate(bufs):
            j = 2 * k + b

            @pl.when(k > 0)
            def _():
                pltpu.make_async_copy(rs, g_hbm.at[pl.ds(0, CHUNK)],
                                      ssem).wait()

            pltpu.async_copy(ts_hbm.at[idx_s.at[j]], rs, lgs)
            pltpu.async_copy(tt_hbm.at[idx_t.at[j]], rt, lgs)
        for b, (rs, rt, lgs, ssem) in enumerate(bufs):
            j = 2 * k + b
            pltpu.make_async_copy(ts_hbm.at[idx_s.at[j]], rs, lgs).wait()
            pltpu.make_async_copy(tt_hbm.at[idx_t.at[j]], rt, lgs).wait()

            def inner(r, carry2):
                for rr in range(2):
                    for gidx in range(H // 16):
                        sl = pl.ds(gidx * 16, 16)
                        rs[2 * r + rr, sl] = (rs[2 * r + rr, sl]
                                              + rt[2 * r + rr, sl])
                return carry2

            lax.fori_loop(0, CHUNK // 2, inner, 0)
            pltpu.async_copy(rs, g_hbm.at[pl.ds(base + j * CHUNK, CHUNK)],
                             ssem)
        return carry

    lax.fori_loop(0, NCHUNK // 2, body, 0)
    for rs, rt, lgs, ssem in bufs:
        pltpu.make_async_copy(rs, g_hbm.at[pl.ds(0, CHUNK)], ssem).wait()


def _scatter_body(z_hbm, idxt_hbm, zeros_hbm, out_hbm, idx_t, zv0, zv1,
                  acc, lz0, lz1, sc0, sc1):
    c = lax.axis_index("c")
    s = lax.axis_index("s")

    @pl.when(s == 0)
    def _():
        pltpu.sync_copy(zeros_hbm, acc)

    plsc.subcore_barrier()
    base = (c * NSUB + s) * EPW
    pltpu.sync_copy(idxt_hbm.at[c, s], idx_t)
    bufs = ((zv0, lz0, sc0), (zv1, lz1, sc1))

    def body(k, carry):
        for b, (zv, lzs, scs) in enumerate(bufs):
            j = 2 * k + b

            @pl.when(k > 0)
            def _():
                pltpu.make_async_copy(zv, acc.at[idx_t.at[j]],
                                      scs).wait()

            pltpu.async_copy(z_hbm.at[pl.ds(base + j * CHUNK, CHUNK)], zv,
                             lzs)
        for b, (zv, lzs, scs) in enumerate(bufs):
            j = 2 * k + b
            pltpu.make_async_copy(z_hbm.at[pl.ds(0, CHUNK)], zv, lzs).wait()
            pltpu.async_copy(zv, acc.at[idx_t.at[j]], scs,
                             add=True)
        return carry

    lax.fori_loop(0, NCHUNK // 2, body, 0)
    for zv, lzs, scs in bufs:
        pltpu.make_async_copy(zv, acc.at[idx_t.at[0]],
                              scs).wait()
    plsc.subcore_barrier()

    @pl.when(s == 0)
    def _():
        pltpu.sync_copy(acc, out_hbm.at[c])


def _counts_body(idxt_hbm, zeros_hbm, ones_hbm, out_hbm, idx_t, onev, acc,
                 csem):
    c = lax.axis_index("c")
    s = lax.axis_index("s")

    @pl.when(s == 0)
    def _():
        pltpu.sync_copy(zeros_hbm, acc)

    pltpu.sync_copy(ones_hbm, onev)
    plsc.subcore_barrier()
    pltpu.sync_copy(idxt_hbm.at[c, s], idx_t)

    def body(j, carry):
        pltpu.async_copy(onev, acc.at[idx_t.at[j]], csem, add=True)
        return carry

    lax.fori_loop(0, NCHUNK, body, 0)

    def drain(j, carry):
        pltpu.make_async_copy(onev, acc.at[idx_t.at[0]], csem).wait()
        return carry

    lax.fori_loop(0, NCHUNK, drain, 0)
    plsc.subcore_barrier()

    @pl.when(s == 0)
    def _():
        pltpu.sync_copy(acc, out_hbm.at[c])


@functools.lru_cache(maxsize=None)
def _sc_kernels():
    mesh = plsc.VectorSubcoreMesh(core_axis_name="c", subcore_axis_name="s",
                                  num_cores=NCORE, num_subcores=NSUB)
    gather = pl.kernel(
        _gather_body,
        out_type=jax.ShapeDtypeStruct((EPAD, H), F32),
        mesh=mesh,
        scratch_types=[
            pltpu.VMEM((NCHUNK, CHUNK), jnp.int32),
            pltpu.VMEM((NCHUNK, CHUNK), jnp.int32),
            pltpu.VMEM((CHUNK, H), F32),
            pltpu.VMEM((CHUNK, H), F32),
            pltpu.VMEM((CHUNK, H), F32),
            pltpu.VMEM((CHUNK, H), F32),
            pltpu.SemaphoreType.DMA,
            pltpu.SemaphoreType.DMA,
            pltpu.SemaphoreType.DMA,
            pltpu.SemaphoreType.DMA,
        ],
    )
    scatter = pl.kernel(
        _scatter_body,
        out_type=jax.ShapeDtypeStruct((NCORE, ACCR, H), F32),
        mesh=mesh,
        scratch_types=[
            pltpu.VMEM((NCHUNK, CHUNK), jnp.int32),
            pltpu.VMEM((CHUNK, H), F32),
            pltpu.VMEM((CHUNK, H), F32),
            pltpu.VMEM_SHARED((ACCR, H), F32),
            pltpu.SemaphoreType.DMA,
            pltpu.SemaphoreType.DMA,
            pltpu.SemaphoreType.DMA,
            pltpu.SemaphoreType.DMA,
        ],
    )
    counts = pl.kernel(
        _counts_body,
        out_type=jax.ShapeDtypeStruct((NCORE, ACCR, H), F32),
        mesh=mesh,
        scratch_types=[
            pltpu.VMEM((NCHUNK, CHUNK), jnp.int32),
            pltpu.VMEM((CHUNK, H), F32),
            pltpu.VMEM_SHARED((ACCR, H), F32),
            pltpu.SemaphoreType.DMA,
        ],
    )
    return gather, scatter, counts


def _gather_sc(ts, tt, idxs, idxt):
    return _sc_kernels()[0](ts, tt, idxs, idxt)


def _scatter_sc(z, idxt):
    return _sc_kernels()[1](z, idxt, jnp.zeros((ACCR, H), F32))


def _counts_sc(idxt):
    return _sc_kernels()[2](idxt, jnp.zeros((ACCR, H), F32),
                            jnp.ones((CHUNK, H), F32))


def _prep_idx(ei):
    pad = EPAD - E
    src = ei[0].astype(jnp.int32)
    tgt = ei[1].astype(jnp.int32)
    shape = (NCORE, NSUB, NCHUNK, CHUNK)
    src_g = jnp.pad(src, (0, pad)).reshape(shape)
    tgt_g = jnp.pad(tgt, (0, pad)).reshape(shape)
    tgt_s = jnp.pad(tgt, (0, pad), constant_values=NV).reshape(shape)
    return src_g, tgt_g, tgt_s


def _bipartite(x_src, x_tgt, src_g, tgt_g, tgt_s, cntp, q, p):
    w1 = p["nn"]["l1"]["W"]
    p_src, p_tgt = _proj(x_src, x_tgt, w1[H:2 * H], w1[0:H])
    g = _gather_sc(p_src, p_tgt, src_g, tgt_g)
    z, stats = _message(g, q, p["nn"]["l2"]["W"],
                        p["nn"]["l2"]["b"].reshape(1, H))
    accp = _scatter_sc(z, tgt_s)
    return _post(accp, cntp, stats, p["nn"]["bn"])


def kernel(var_node_features, con_node_features, edge_index_var,
           edge_index_con, edge_features_var, edge_features_con,
           num_nodes_var, num_nodes_con, params):
    del num_nodes_var, num_nodes_con
    src_gv, tgt_gv, tgt_sv = _prep_idx(edge_index_var)
    src_gc, tgt_gc, tgt_sc_ = _prep_idx(edge_index_con)
    ef_var = jnp.pad(edge_features_var.astype(F32), ((0, EPAD - E), (0, 0)))
    ef_con = jnp.pad(edge_features_con.astype(F32), ((0, EPAD - E), (0, 0)))

    xv = _encoder(var_node_features.astype(F32), params["var_enc"])
    xc = _encoder(con_node_features.astype(F32), params["con_enc"])

    cntp_c = _counts_sc(tgt_sv)
    cntp_v = _counts_sc(tgt_sc_)

    qs_var = [_edge_q(ef_var, p["edge_encoder"], p["nn"]["l1"]["W"][2 * H:],
                      p["nn"]["l1"]["b"]) for p in params["layers_var"]]
    qs_con = [_edge_q(ef_con, p["edge_encoder"], p["nn"]["l1"]["W"][2 * H:],
                      p["nn"]["l1"]["b"]) for p in params["layers_con"]]

    x_var = [xv]
    x_con = [xc]
    for i in range(2):
        x_con.append(_bipartite(x_var[-1], x_con[-1], src_gv, tgt_gv, tgt_sv,
                                cntp_c, qs_var[i], params["layers_var"][i]))
        x_var.append(_bipartite(x_con[-1], x_var[-1], src_gc, tgt_gc, tgt_sc_,
                                cntp_v, qs_con[i], params["layers_con"][i]))

    x = jnp.concatenate(x_var, axis=-1)
    return _head(x, params)


# speedup vs baseline: 1.4295x; 1.0217x over previous
import functools

import jax
import jax.numpy as jnp
from jax import lax
from jax.experimental import pallas as pl
from jax.experimental.pallas import tpu as pltpu
from jax.experimental.pallas import tpu_sc as plsc

F32 = jnp.float32
H = 128
NV = 5000
E = 160000
NCORE = 2
NSUB = 16
CHUNK = 128
NCHUNK = 40
EPW = CHUNK * NCHUNK
EPAD = NCORE * NSUB * EPW
ACCR = 5120
MTILE = 512
NMT = EPAD // MTILE
BNEPS = 1e-5


def _dotf(a, b):
    return jnp.dot(a, b, preferred_element_type=F32)


def _enc_body(x_ref, w1_ref, b1_ref, w2_ref, b2_ref, o_ref):
    h = jnp.maximum(_dotf(x_ref[...], w1_ref[...]) + b1_ref[...], 0.0)
    o_ref[...] = _dotf(h, w2_ref[...]) + b2_ref[...]


def _encoder(x, p):
    return pl.pallas_call(
        _enc_body,
        out_shape=jax.ShapeDtypeStruct((NV, H), F32),
    )(x, p["l1"]["W"], p["l1"]["b"].reshape(1, H),
      p["l2"]["W"], p["l2"]["b"].reshape(1, H))


def _proj_body(xs_ref, xt_ref, ws_ref, wt_ref, os_ref, ot_ref):
    os_ref[...] = _dotf(xs_ref[...], ws_ref[...])
    ot_ref[...] = _dotf(xt_ref[...], wt_ref[...])


def _proj(x_src, x_tgt, w_src, w_tgt):
    return pl.pallas_call(
        _proj_body,
        out_shape=[jax.ShapeDtypeStruct((NV, H), F32),
                   jax.ShapeDtypeStruct((NV, H), F32)],
    )(x_src, x_tgt, w_src, w_tgt)


def _edge_z(ef, w1, b1, w2, b2):
    h1 = jnp.maximum(ef * w1 + b1, 0.0)
    return jnp.maximum(_dotf(h1, w2) + b2, 0.0)


def _estats_body(ef_ref, w1_ref, b1_ref, w2_ref, b2_ref, o_ref):
    i = pl.program_id(0)
    z = _edge_z(ef_ref[...], w1_ref[...], b1_ref[...], w2_ref[...], b2_ref[...])
    rid = lax.broadcasted_iota(jnp.int32, (MTILE, 1), 0) + i * MTILE
    msk = rid < E
    s0 = jnp.sum(jnp.where(msk, z, 0.0), axis=0, keepdims=True)
    s1 = jnp.sum(jnp.where(msk, z * z, 0.0), axis=0, keepdims=True)

    @pl.when(i == 0)
    def _():
        o_ref[...] = jnp.zeros_like(o_ref)

    o_ref[...] += jnp.concatenate([s0, s1], axis=0)


def _bn_affine(stats, gamma, beta):
    mean = stats[0:1] / E
    var = stats[1:2] / E - mean * mean
    a = gamma * lax.rsqrt(var + BNEPS)
    return a, beta - mean * a


def _eq_body(ef_ref, w1_ref, b1_ref, w2_ref, b2_ref, st_ref, g_ref, bt_ref,
             we_ref, bn1_ref, o_ref):
    z = _edge_z(ef_ref[...], w1_ref[...], b1_ref[...], w2_ref[...], b2_ref[...])
    a, c = _bn_affine(st_ref[...], g_ref[...], bt_ref[...])
    o_ref[...] = _dotf(z * a + c, we_ref[...]) + bn1_ref[...]


def _edge_q(ef_pad, enc_p, w_e, b_nn1):
    w1 = enc_p["l1"]["W"]
    b1 = enc_p["l1"]["b"].reshape(1, H)
    w2 = enc_p["l2"]["W"]
    b2 = enc_p["l2"]["b"].reshape(1, H)
    gamma = enc_p["bn"]["gamma"].reshape(1, H)
    beta = enc_p["bn"]["beta"].reshape(1, H)
    wspec = pl.BlockSpec((1, H), lambda i: (0, 0))
    wwspec = pl.BlockSpec((H, H), lambda i: (0, 0))
    efspec = pl.BlockSpec((MTILE, 1), lambda i: (i, 0))
    stats = pl.pallas_call(
        _estats_body,
        grid=(NMT,),
        in_specs=[efspec, wspec, wspec, wwspec, wspec],
        out_specs=pl.BlockSpec((2, H), lambda i: (0, 0)),
        out_shape=jax.ShapeDtypeStruct((2, H), F32),
    )(ef_pad, w1, b1, w2, b2)
    return pl.pallas_call(
        _eq_body,
        grid=(NMT,),
        in_specs=[efspec, wspec, wspec, wwspec, wspec,
                  pl.BlockSpec((2, H), lambda i: (0, 0)), wspec, wspec,
                  wwspec, wspec],
        out_specs=pl.BlockSpec((MTILE, H), lambda i: (i, 0)),
        out_shape=jax.ShapeDtypeStruct((EPAD, H), F32),
    )(ef_pad, w1, b1, w2, b2, stats, gamma, beta, w_e, b_nn1.reshape(1, H))


def _msg_body(g_ref, q_ref, w2_ref, b2_ref, z_ref, st_ref):
    i = pl.program_id(0)
    h = jnp.maximum(g_ref[...] + q_ref[...], 0.0)
    z = jnp.maximum(_dotf(h, w2_ref[...]) + b2_ref[...], 0.0)
    z_ref[...] = z
    rid = lax.broadcasted_iota(jnp.int32, (MTILE, 1), 0) + i * MTILE
    msk = rid < E
    s0 = jnp.sum(jnp.where(msk, z, 0.0), axis=0, keepdims=True)
    s1 = jnp.sum(jnp.where(msk, z * z, 0.0), axis=0, keepdims=True)

    @pl.when(i == 0)
    def _():
        st_ref[...] = jnp.zeros_like(st_ref)

    st_ref[...] += jnp.concatenate([s0, s1], axis=0)


def _message(g, q, w2, b2):
    return pl.pallas_call(
        _msg_body,
        grid=(NMT,),
        in_specs=[pl.BlockSpec((MTILE, H), lambda i: (i, 0)),
                  pl.BlockSpec((MTILE, H), lambda i: (i, 0)),
                  pl.BlockSpec((H, H), lambda i: (0, 0)),
                  pl.BlockSpec((1, H), lambda i: (0, 0))],
        out_specs=[pl.BlockSpec((MTILE, H), lambda i: (i, 0)),
                   pl.BlockSpec((2, H), lambda i: (0, 0))],
        out_shape=[jax.ShapeDtypeStruct((EPAD, H), F32),
                   jax.ShapeDtypeStruct((2, H), F32)],
    )(g, q, w2, b2)


def _post_body(acc_ref, cnt_ref, st_ref, g_ref, bt_ref, o_ref):
    s = acc_ref[0, :NV, :] + acc_ref[1, :NV, :]
    cnt = cnt_ref[0, :NV, 0:1] + cnt_ref[1, :NV, 0:1]
    a, c = _bn_affine(st_ref[...], g_ref[...], bt_ref[...])
    o_ref[...] = jnp.maximum((s * a + cnt * c) / jnp.maximum(cnt, 1.0), 0.0)


def _post(accp, cntp, stats, bn_p):
    return pl.pallas_call(
        _post_body,
        out_shape=jax.ShapeDtypeStruct((NV, H), F32),
    )(accp, cntp, stats, bn_p["gamma"].reshape(1, H), bn_p["beta"].reshape(1, H))


def _head_body(x_ref, w1_ref, b1_ref, w2_ref, b2_ref, w3_ref, b3_ref,
               w4_ref, b4_ref, o_ref):
    h = jnp.maximum(_dotf(x_ref[...], w1_ref[...]) + b1_ref[...], 0.0)
    h = jnp.maximum(_dotf(h, w2_ref[...]) + b2_ref[...], 0.0)
    h = jnp.maximum(_dotf(h, w3_ref[...]) + b3_ref[...], 0.0)
    o = _dotf(h, w4_ref[...]) + b4_ref[...]
    m = jnp.max(o, axis=-1, keepdims=True)
    lse = m + jnp.log(jnp.sum(jnp.exp(o - m), axis=-1, keepdims=True))
    o_ref[...] = o - lse


def _head(x, params):
    args = [x]
    for name in ("lin1", "lin2", "lin3", "lin4"):
        args.append(params[name]["W"])
        args.append(params[name]["b"].reshape(1, -1))
    return pl.pallas_call(
        _head_body,
        out_shape=jax.ShapeDtypeStruct((NV, 2), F32),
    )(*args)


def _gather_body(ts_hbm, tt_hbm, idxs_hbm, idxt_hbm, g_hbm,
                 idx_s, idx_t, rs0, rs1, rt0, rt1, go0, go1,
                 lg0, lg1, ss0, ss1):
    c = lax.axis_index("c")
    s = lax.axis_index("s")
    base = (c * NSUB + s) * EPW
    pltpu.sync_copy(idxs_hbm.at[c, s], idx_s)
    pltpu.sync_copy(idxt_hbm.at[c, s], idx_t)
    bufs = ((rs0, rt0, go0, lg0, ss0), (rs1, rt1, go1, lg1, ss1))

    def _fire(j, rs, rt, lgs):
        pltpu.async_copy(ts_hbm.at[idx_s.at[j]], rs, lgs)
        pltpu.async_copy(tt_hbm.at[idx_t.at[j]], rt, lgs)

    for b, (rs, rt, go, lgs, ssem) in enumerate(bufs):
        _fire(b, rs, rt, lgs)

    def body(k, carry):
        for b, (rs, rt, go, lgs, ssem) in enumerate(bufs):
            j = 2 * k + b
            pltpu.make_async_copy(ts_hbm.at[idx_s.at[j]], rs, lgs).wait()
            pltpu.make_async_copy(tt_hbm.at[idx_t.at[j]], rt, lgs).wait()

            @pl.when(k > 0)
            def _():
                pltpu.make_async_copy(go, g_hbm.at[pl.ds(0, CHUNK)],
                                      ssem).wait()

            def inner(r, carry2):
                for rr in range(2):
                    for gidx in range(H // 16):
                        sl = pl.ds(gidx * 16, 16)
                        go[2 * r + rr, sl] = (rs[2 * r + rr, sl]
                                              + rt[2 * r + rr, sl])
                return carry2

            lax.fori_loop(0, CHUNK // 2, inner, 0)
            pltpu.async_copy(go, g_hbm.at[pl.ds(base + j * CHUNK, CHUNK)],
                             ssem)

            @pl.when(j + 2 < NCHUNK)
            def _():
                _fire(j + 2, rs, rt, lgs)

        return carry

    lax.fori_loop(0, NCHUNK // 2, body, 0)
    for rs, rt, go, lgs, ssem in bufs:
        pltpu.make_async_copy(go, g_hbm.at[pl.ds(0, CHUNK)], ssem).wait()


def _scatter_body(z_hbm, idxt_hbm, zeros_hbm, out_hbm, idx_t, zv0, zv1,
                  acc, lz0, lz1, sc0, sc1):
    c = lax.axis_index("c")
    s = lax.axis_index("s")

    @pl.when(s == 0)
    def _():
        pltpu.sync_copy(zeros_hbm, acc)

    plsc.subcore_barrier()
    base = (c * NSUB + s) * EPW
    pltpu.sync_copy(idxt_hbm.at[c, s], idx_t)
    bufs = ((zv0, lz0, sc0), (zv1, lz1, sc1))

    def body(k, carry):
        for b, (zv, lzs, scs) in enumerate(bufs):
            j = 2 * k + b

            @pl.when(k > 0)
            def _():
                pltpu.make_async_copy(zv, acc.at[idx_t.at[j]],
                                      scs).wait()

            pltpu.async_copy(z_hbm.at[pl.ds(base + j * CHUNK, CHUNK)], zv,
                             lzs)
        for b, (zv, lzs, scs) in enumerate(bufs):
            j = 2 * k + b
            pltpu.make_async_copy(z_hbm.at[pl.ds(0, CHUNK)], zv, lzs).wait()
            pltpu.async_copy(zv, acc.at[idx_t.at[j]], scs,
                             add=True)
        return carry

    lax.fori_loop(0, NCHUNK // 2, body, 0)
    for zv, lzs, scs in bufs:
        pltpu.make_async_copy(zv, acc.at[idx_t.at[0]],
                              scs).wait()
    plsc.subcore_barrier()

    @pl.when(s == 0)
    def _():
        pltpu.sync_copy(acc, out_hbm.at[c])


def _counts_body(idxt_hbm, zeros_hbm, ones_hbm, out_hbm, idx_t, onev, acc,
                 csem):
    c = lax.axis_index("c")
    s = lax.axis_index("s")

    @pl.when(s == 0)
    def _():
        pltpu.sync_copy(zeros_hbm, acc)

    pltpu.sync_copy(ones_hbm, onev)
    plsc.subcore_barrier()
    pltpu.sync_copy(idxt_hbm.at[c, s], idx_t)

    def body(j, carry):
        pltpu.async_copy(onev, acc.at[idx_t.at[j]], csem, add=True)
        return carry

    lax.fori_loop(0, NCHUNK, body, 0)

    def drain(j, carry):
        pltpu.make_async_copy(onev, acc.at[idx_t.at[0]], csem).wait()
        return carry

    lax.fori_loop(0, NCHUNK, drain, 0)
    plsc.subcore_barrier()

    @pl.when(s == 0)
    def _():
        pltpu.sync_copy(acc, out_hbm.at[c])


@functools.lru_cache(maxsize=None)
def _sc_kernels():
    mesh = plsc.VectorSubcoreMesh(core_axis_name="c", subcore_axis_name="s",
                                  num_cores=NCORE, num_subcores=NSUB)
    gather = pl.kernel(
        _gather_body,
        out_type=jax.ShapeDtypeStruct((EPAD, H), F32),
        mesh=mesh,
        scratch_types=[
            pltpu.VMEM((NCHUNK, CHUNK), jnp.int32),
            pltpu.VMEM((NCHUNK, CHUNK), jnp.int32),
            pltpu.VMEM((CHUNK, H), F32),
            pltpu.VMEM((CHUNK, H), F32),
            pltpu.VMEM((CHUNK, H), F32),
            pltpu.VMEM((CHUNK, H), F32),
            pltpu.VMEM((CHUNK, H), F32),
            pltpu.VMEM((CHUNK, H), F32),
            pltpu.SemaphoreType.DMA,
            pltpu.SemaphoreType.DMA,
            pltpu.SemaphoreType.DMA,
            pltpu.SemaphoreType.DMA,
        ],
    )
    scatter = pl.kernel(
        _scatter_body,
        out_type=jax.ShapeDtypeStruct((NCORE, ACCR, H), F32),
        mesh=mesh,
        scratch_types=[
            pltpu.VMEM((NCHUNK, CHUNK), jnp.int32),
            pltpu.VMEM((CHUNK, H), F32),
            pltpu.VMEM((CHUNK, H), F32),
            pltpu.VMEM_SHARED((ACCR, H), F32),
            pltpu.SemaphoreType.DMA,
            pltpu.SemaphoreType.DMA,
            pltpu.SemaphoreType.DMA,
            pltpu.SemaphoreType.DMA,
        ],
    )
    counts = pl.kernel(
        _counts_body,
        out_type=jax.ShapeDtypeStruct((NCORE, ACCR, H), F32),
        mesh=mesh,
        scratch_types=[
            pltpu.VMEM((NCHUNK, CHUNK), jnp.int32),
            pltpu.VMEM((CHUNK, H), F32),
            pltpu.VMEM_SHARED((ACCR, H), F32),
            pltpu.SemaphoreType.DMA,
        ],
    )
    return gather, scatter, counts


def _gather_sc(ts, tt, idxs, idxt):
    return _sc_kernels()[0](ts, tt, idxs, idxt)


def _scatter_sc(z, idxt):
    return _sc_kernels()[1](z, idxt, jnp.zeros((ACCR, H), F32))


def _counts_sc(idxt):
    return _sc_kernels()[2](idxt, jnp.zeros((ACCR, H), F32),
                            jnp.ones((CHUNK, H), F32))


def _prep_idx(ei):
    pad = EPAD - E
    src = ei[0].astype(jnp.int32)
    tgt = ei[1].astype(jnp.int32)
    shape = (NCORE, NSUB, NCHUNK, CHUNK)
    src_g = jnp.pad(src, (0, pad)).reshape(shape)
    tgt_g = jnp.pad(tgt, (0, pad)).reshape(shape)
    tgt_s = jnp.pad(tgt, (0, pad), constant_values=NV).reshape(shape)
    return src_g, tgt_g, tgt_s


def _bipartite(x_src, x_tgt, src_g, tgt_g, tgt_s, cntp, q, p):
    w1 = p["nn"]["l1"]["W"]
    p_src, p_tgt = _proj(x_src, x_tgt, w1[H:2 * H], w1[0:H])
    g = _gather_sc(p_src, p_tgt, src_g, tgt_g)
    z, stats = _message(g, q, p["nn"]["l2"]["W"],
                        p["nn"]["l2"]["b"].reshape(1, H))
    accp = _scatter_sc(z, tgt_s)
    return _post(accp, cntp, stats, p["nn"]["bn"])


def kernel(var_node_features, con_node_features, edge_index_var,
           edge_index_con, edge_features_var, edge_features_con,
           num_nodes_var, num_nodes_con, params):
    del num_nodes_var, num_nodes_con
    src_gv, tgt_gv, tgt_sv = _prep_idx(edge_index_var)
    src_gc, tgt_gc, tgt_sc_ = _prep_idx(edge_index_con)
    ef_var = jnp.pad(edge_features_var.astype(F32), ((0, EPAD - E), (0, 0)))
    ef_con = jnp.pad(edge_features_con.astype(F32), ((0, EPAD - E), (0, 0)))

    xv = _encoder(var_node_features.astype(F32), params["var_enc"])
    xc = _encoder(con_node_features.astype(F32), params["con_enc"])

    cntp_c = _counts_sc(tgt_sv)
    cntp_v = _counts_sc(tgt_sc_)

    qs_var = [_edge_q(ef_var, p["edge_encoder"], p["nn"]["l1"]["W"][2 * H:],
                      p["nn"]["l1"]["b"]) for p in params["layers_var"]]
    qs_con = [_edge_q(ef_con, p["edge_encoder"], p["nn"]["l1"]["W"][2 * H:],
                      p["nn"]["l1"]["b"]) for p in params["layers_con"]]

    x_var = [xv]
    x_con = [xc]
    for i in range(2):
        x_con.append(_bipartite(x_var[-1], x_con[-1], src_gv, tgt_gv, tgt_sv,
                                cntp_c, qs_var[i], params["layers_var"][i]))
        x_var.append(_bipartite(x_con[-1], x_var[-1], src_gc, tgt_gc, tgt_sc_,
                                cntp_v, qs_con[i], params["layers_con"][i]))

    x = jnp.concatenate(x_var, axis=-1)
    return _head(x, params)
